# baseline (speedup 1.0000x reference)
import functools

import jax
import jax.numpy as jnp
from jax import lax
from jax.experimental import pallas as pl
from jax.experimental.pallas import tpu as pltpu, tpu_sc as plsc

_info = plsc.get_sparse_core_info()
_NC, _NS = _info.num_cores, _info.num_subcores
_NW = _NC * _NS

_VROWS = 104
_IROWS = 208

_mesh = plsc.VectorSubcoreMesh(core_axis_name="c", subcore_axis_name="s")


@functools.partial(
    pl.kernel,
    mesh=_mesh,
    out_type=(
        jax.ShapeDtypeStruct((3328, 128), jnp.float32),
        jax.ShapeDtypeStruct((6656, 128), jnp.int32),
    ),
    scratch_types=[
        pltpu.VMEM((_VROWS, 128), jnp.float32),
        pltpu.VMEM((_IROWS, 128), jnp.int32),
    ],
)
def _sc_copy(vp_hbm, ip_hbm, vo_hbm, io_hbm, v_vmem, i_vmem):
    wid = lax.axis_index("s") * _NC + lax.axis_index("c")
    vbase = wid * _VROWS
    ibase = wid * _IROWS
    pltpu.sync_copy(vp_hbm.at[pl.ds(vbase, _VROWS), :], v_vmem)
    pltpu.sync_copy(v_vmem, vo_hbm.at[pl.ds(vbase, _VROWS), :])
    pltpu.sync_copy(ip_hbm.at[pl.ds(ibase, _IROWS), :], i_vmem)
    pltpu.sync_copy(i_vmem, io_hbm.at[pl.ds(ibase, _IROWS), :])


def kernel(vertices, indices):
    vp = jnp.pad(vertices, ((0, 6496), (0, 1))).reshape(3328, 128)
    ip = jnp.pad(indices, ((0, 12992), (0, 1))).reshape(6656, 128)
    vo, io = _sc_copy(vp, ip)
    v = vo.reshape(106496, 4)[:100000, :3]
    i = io.reshape(212992, 4)[:200000, :3]
    return v, i


# speedup vs baseline: 1.5878x; 1.5878x over previous
import jax
from jax.experimental import pallas as pl

_GRID = 3


def _copy_kernel(v_ref, i_ref, vo_ref, io_ref):
    vo_ref[...] = v_ref[...]
    io_ref[...] = i_ref[...]


def kernel(vertices, indices):
    v2 = vertices.reshape(24, 12500)
    i2 = indices.reshape(48, 12500)
    vo, io = pl.pallas_call(
        _copy_kernel,
        grid=(_GRID,),
        out_shape=(
            jax.ShapeDtypeStruct(v2.shape, v2.dtype),
            jax.ShapeDtypeStruct(i2.shape, i2.dtype),
        ),
        in_specs=[
            pl.BlockSpec((8, 12500), lambda j: (j, 0)),
            pl.BlockSpec((16, 12500), lambda j: (j, 0)),
        ],
        out_specs=(
            pl.BlockSpec((8, 12500), lambda j: (j, 0)),
            pl.BlockSpec((16, 12500), lambda j: (j, 0)),
        ),
    )(v2, i2)
    return vo.reshape(vertices.shape), io.reshape(indices.shape)


# speedup vs baseline: 1.9443x; 1.2245x over previous
import jax
from jax.experimental import pallas as pl

_GRID = 15
_VROWS = 40
_IROWS = 80


def _copy_kernel(v_ref, i_ref, vo_ref, io_ref):
    vo_ref[...] = v_ref[...]
    io_ref[...] = i_ref[...]


def kernel(vertices, indices):
    v2 = vertices.reshape(600, 500)
    i2 = indices.reshape(1200, 500)
    vo, io = pl.pallas_call(
        _copy_kernel,
        grid=(_GRID,),
        out_shape=(
            jax.ShapeDtypeStruct(v2.shape, v2.dtype),
            jax.ShapeDtypeStruct(i2.shape, i2.dtype),
        ),
        in_specs=[
            pl.BlockSpec((_VROWS, 500), lambda j: (j, 0)),
            pl.BlockSpec((_IROWS, 500), lambda j: (j, 0)),
        ],
        out_specs=(
            pl.BlockSpec((_VROWS, 500), lambda j: (j, 0)),
            pl.BlockSpec((_IROWS, 500), lambda j: (j, 0)),
        ),
    )(v2, i2)
    return vo.reshape(vertices.shape), io.reshape(indices.shape)
